# baseline (speedup 1.0000x reference)
import jax
import jax.numpy as jnp
from jax import lax
from jax.experimental import pallas as pl
from jax.experimental.pallas import tpu as pltpu

F = 32
CHUNK = 64
W = 128
NTILE = 7811
TAIL = 999872
TCUT = 999936


def _gather_body(u_sref, i_sref, t_ug, t_ig, t_um, t_im,
                 tl_ug, tl_ig, tl_um, tl_im,
                 o_ug, o_ig, o_um, o_im,
                 s_ug, s_ig, s_um, s_im, cm_u, cm_i,
                 sem_ug, sem_ig, sem_um, sem_im):
    ci = pl.program_id(0)
    srcs = (t_ug, t_ig, t_um, t_im)
    scrs = (s_ug, s_ig, s_um, s_im)
    sems = (sem_ug, sem_ig, sem_um, sem_im)

    def win(r):
        off = pl.multiple_of(jnp.minimum(r >> 7, NTILE) * 128, 128)
        return off, jnp.where(r >= TCUT, r - TAIL, r - off)

    for j in range(CHUNK):
        ru = u_sref[ci * CHUNK + j]
        ri = i_sref[ci * CHUNK + j]
        off_u, cu = win(ru)
        off_i, col_i = win(ri)
        for t in range(4):
            off = off_u if t % 2 == 0 else off_i
            pltpu.make_async_copy(
                srcs[t].at[:, pl.ds(off, W)],
                scrs[t].at[pl.ds(j * F, F), :],
                sems[t]).start()
        cm_u[pl.ds(j * F, F), :] = jnp.full((F, 1), cu, jnp.int32)
        cm_i[pl.ds(j * F, F), :] = jnp.full((F, 1), col_i, jnp.int32)

    for t in range(4):
        for j in range(CHUNK):
            pltpu.make_async_copy(
                srcs[t].at[:, pl.ds(0, W)],
                scrs[t].at[pl.ds(j * F, F), :],
                sems[t]).wait()

    for j in range(CHUNK):
        ru = u_sref[ci * CHUNK + j]
        ri = i_sref[ci * CHUNK + j]

        @pl.when(ru >= TCUT)
        def _():
            s_ug[pl.ds(j * F, F), :] = tl_ug[...]
            s_um[pl.ds(j * F, F), :] = tl_um[...]

        @pl.when(ri >= TCUT)
        def _():
            s_ig[pl.ds(j * F, F), :] = tl_ig[...]
            s_im[pl.ds(j * F, F), :] = tl_im[...]

    u_idx = cm_u[...]
    i_idx = cm_i[...]
    o_ug[...] = jnp.take_along_axis(s_ug[...], u_idx, axis=1)
    o_ig[...] = jnp.take_along_axis(s_ig[...], i_idx, axis=1)
    o_um[...] = jnp.take_along_axis(s_um[...], u_idx, axis=1)
    o_im[...] = jnp.take_along_axis(s_im[...], i_idx, axis=1)


def _gather4(user, item, t_ug, t_ig, t_um, t_im, B):
    n = B // CHUNK
    blk = CHUNK * F
    tbl_spec = pl.BlockSpec(memory_space=pl.ANY)
    tail_spec = pl.BlockSpec((F, W), lambda i, su, si: (0, 0))
    out_spec = pl.BlockSpec((blk, 1), lambda i, su, si: (i, 0))
    grid_spec = pltpu.PrefetchScalarGridSpec(
        num_scalar_prefetch=2,
        grid=(n,),
        in_specs=[tbl_spec] * 4 + [tail_spec] * 4,
        out_specs=[out_spec] * 4,
        scratch_shapes=(
            [pltpu.VMEM((blk, W), jnp.float32) for _ in range(4)]
            + [pltpu.VMEM((blk, 1), jnp.int32) for _ in range(2)]
            + [pltpu.SemaphoreType.DMA for _ in range(4)]
        ),
    )
    tails = [t[:, TAIL:] for t in (t_ug, t_ig, t_um, t_im)]
    outs = pl.pallas_call(
        _gather_body,
        grid_spec=grid_spec,
        out_shape=[jax.ShapeDtypeStruct((B * F, 1), jnp.float32)] * 4,
    )(user, item, t_ug, t_ig, t_um, t_im, *tails)
    return [o.reshape(B, F) for o in outs]


def _mish(x):
    return x * jnp.tanh(jax.nn.softplus(x))


def _mlp_body(xr, w0a, w0b, b0r, w1, b1r, w2, b2r, wpa, wpb, bpr, out):
    x = xr[...]
    eu_m = x[:, 2 * F:3 * F]
    ei_m = x[:, 3 * F:4 * F]
    h = jnp.dot(eu_m, w0a[...]) + jnp.dot(ei_m, w0b[...]) + b0r[...]
    h = _mish(h)
    h = _mish(jnp.dot(h, w1[...]) + b1r[...])
    h = _mish(jnp.dot(h, w2[...]) + b2r[...])
    g = x[:, 0:F] * x[:, F:2 * F]
    p = (jnp.sum(g * wpa[...], axis=1, keepdims=True)
         + jnp.sum(h * wpb[...], axis=1, keepdims=True) + bpr[...])
    out[...] = _mish(p)


def _mlp_tc(xg, W0, b0, W1, b1, W2, b2, Wp, bp, B):
    blk = 2048
    grid = (B // blk,)
    w0a = W0[:F]
    w0b = W0[F:]
    wpa = Wp[:F].reshape(1, F)
    wpb = Wp[F:].reshape(1, F)
    b0r = b0.reshape(1, -1)
    b1r = b1.reshape(1, -1)
    b2r = b2.reshape(1, -1)
    bpr = bp.reshape(1, 1)

    def full_spec(a):
        return pl.BlockSpec(a.shape, lambda i: (0,) * a.ndim)

    out = pl.pallas_call(
        _mlp_body,
        grid=grid,
        in_specs=[
            pl.BlockSpec((blk, 4 * F), lambda i: (i, 0)),
            full_spec(w0a), full_spec(w0b), full_spec(b0r),
            full_spec(W1), full_spec(b1r),
            full_spec(W2), full_spec(b2r),
            full_spec(wpa), full_spec(wpb), full_spec(bpr),
        ],
        out_specs=pl.BlockSpec((blk, 1), lambda i: (i, 0)),
        out_shape=jax.ShapeDtypeStruct((B, 1), jnp.float32),
    )(xg, w0a, w0b, b0r, W1, b1r, W2, b2r, wpa, wpb, bpr)
    return out.reshape(-1)


def kernel(user, item, embed_user_GMF, embed_item_GMF, embed_user_MLP,
           embed_item_MLP, W0, b0, W1, b1, W2, b2, Wp, bp):
    B = user.shape[0]
    u32 = user.astype(jnp.int32)
    i32 = item.astype(jnp.int32)
    eg, ig, em, im = _gather4(u32, i32, embed_user_GMF.T, embed_item_GMF.T,
                              embed_user_MLP.T, embed_item_MLP.T, B)
    xg = jnp.concatenate([eg, ig, em, im], axis=1)
    return _mlp_tc(xg, W0, b0, W1, b1, W2, b2, Wp, bp, B)


# speedup vs baseline: 1.3932x; 1.3932x over previous
import jax
import jax.numpy as jnp
from jax import lax
from jax.experimental import pallas as pl
from jax.experimental.pallas import tpu as pltpu

F = 32
CHUNK = 128
W = 128
NTILE = 7811
TAIL = 999872
TCUT = 999936


def _gather_body(u_sref, i_sref, t_ug, t_ig, t_um, t_im,
                 tl_ug, tl_ig, tl_um, tl_im,
                 o_ug, o_ig, o_um, o_im,
                 s_ug, s_ig, s_um, s_im, cm_u, cm_i,
                 sem_ug, sem_ig, sem_um, sem_im):
    ci = pl.program_id(0)
    srcs = (t_ug, t_ig, t_um, t_im)
    scrs = (s_ug, s_ig, s_um, s_im)
    sems = (sem_ug, sem_ig, sem_um, sem_im)

    def win(r):
        off = pl.multiple_of(jnp.minimum(r >> 7, NTILE) * 128, 128)
        return off, jnp.where(r >= TCUT, r - TAIL, r - off)

    for j in range(CHUNK):
        ru = u_sref[ci * CHUNK + j]
        ri = i_sref[ci * CHUNK + j]
        off_u, cu = win(ru)
        off_i, col_i = win(ri)
        for t in range(4):
            off = off_u if t % 2 == 0 else off_i
            pltpu.make_async_copy(
                srcs[t].at[:, pl.ds(off, W)],
                scrs[t].at[pl.ds(j * F, F), :],
                sems[t]).start()
        cm_u[pl.ds(j * F, F), :] = jnp.full((F, 1), cu, jnp.int32)
        cm_i[pl.ds(j * F, F), :] = jnp.full((F, 1), col_i, jnp.int32)

    for t in range(4):
        for j in range(CHUNK):
            pltpu.make_async_copy(
                srcs[t].at[:, pl.ds(0, W)],
                scrs[t].at[pl.ds(j * F, F), :],
                sems[t]).wait()

    for j in range(CHUNK):
        ru = u_sref[ci * CHUNK + j]
        ri = i_sref[ci * CHUNK + j]

        @pl.when(ru >= TCUT)
        def _():
            s_ug[pl.ds(j * F, F), :] = tl_ug[...]
            s_um[pl.ds(j * F, F), :] = tl_um[...]

        @pl.when(ri >= TCUT)
        def _():
            s_ig[pl.ds(j * F, F), :] = tl_ig[...]
            s_im[pl.ds(j * F, F), :] = tl_im[...]

    lane = lax.broadcasted_iota(jnp.int32, (CHUNK * F, W), 1)
    ones = jnp.ones((W, 1), jnp.float32)
    mask_u = lane == cm_u[...]
    mask_i = lane == cm_i[...]
    o_ug[...] = jnp.dot(jnp.where(mask_u, s_ug[...], 0.0), ones)
    o_ig[...] = jnp.dot(jnp.where(mask_i, s_ig[...], 0.0), ones)
    o_um[...] = jnp.dot(jnp.where(mask_u, s_um[...], 0.0), ones)
    o_im[...] = jnp.dot(jnp.where(mask_i, s_im[...], 0.0), ones)


def _gather4(user, item, t_ug, t_ig, t_um, t_im, B):
    n = B // CHUNK
    blk = CHUNK * F
    tbl_spec = pl.BlockSpec(memory_space=pl.ANY)
    tail_spec = pl.BlockSpec((F, W), lambda i, su, si: (0, 0))
    out_spec = pl.BlockSpec((blk, 1), lambda i, su, si: (i, 0))
    grid_spec = pltpu.PrefetchScalarGridSpec(
        num_scalar_prefetch=2,
        grid=(n,),
        in_specs=[tbl_spec] * 4 + [tail_spec] * 4,
        out_specs=[out_spec] * 4,
        scratch_shapes=(
            [pltpu.VMEM((blk, W), jnp.float32) for _ in range(4)]
            + [pltpu.VMEM((blk, 1), jnp.int32) for _ in range(2)]
            + [pltpu.SemaphoreType.DMA for _ in range(4)]
        ),
    )
    tails = [t[:, TAIL:] for t in (t_ug, t_ig, t_um, t_im)]
    outs = pl.pallas_call(
        _gather_body,
        grid_spec=grid_spec,
        out_shape=[jax.ShapeDtypeStruct((B * F, 1), jnp.float32)] * 4,
    )(user, item, t_ug, t_ig, t_um, t_im, *tails)
    return [o.reshape(B, F) for o in outs]


def _mish(x):
    return x * jnp.tanh(jax.nn.softplus(x))


def _mlp_body(xr, w0a, w0b, b0r, w1, b1r, w2, b2r, wpa, wpb, bpr, out):
    x = xr[...]
    eu_m = x[:, 2 * F:3 * F]
    ei_m = x[:, 3 * F:4 * F]
    h = jnp.dot(eu_m, w0a[...]) + jnp.dot(ei_m, w0b[...]) + b0r[...]
    h = _mish(h)
    h = _mish(jnp.dot(h, w1[...]) + b1r[...])
    h = _mish(jnp.dot(h, w2[...]) + b2r[...])
    g = x[:, 0:F] * x[:, F:2 * F]
    p = (jnp.sum(g * wpa[...], axis=1, keepdims=True)
         + jnp.sum(h * wpb[...], axis=1, keepdims=True) + bpr[...])
    out[...] = _mish(p)


def _mlp_tc(xg, W0, b0, W1, b1, W2, b2, Wp, bp, B):
    blk = 2048
    grid = (B // blk,)
    w0a = W0[:F]
    w0b = W0[F:]
    wpa = Wp[:F].reshape(1, F)
    wpb = Wp[F:].reshape(1, F)
    b0r = b0.reshape(1, -1)
    b1r = b1.reshape(1, -1)
    b2r = b2.reshape(1, -1)
    bpr = bp.reshape(1, 1)

    def full_spec(a):
        return pl.BlockSpec(a.shape, lambda i: (0,) * a.ndim)

    out = pl.pallas_call(
        _mlp_body,
        grid=grid,
        in_specs=[
            pl.BlockSpec((blk, 4 * F), lambda i: (i, 0)),
            full_spec(w0a), full_spec(w0b), full_spec(b0r),
            full_spec(W1), full_spec(b1r),
            full_spec(W2), full_spec(b2r),
            full_spec(wpa), full_spec(wpb), full_spec(bpr),
        ],
        out_specs=pl.BlockSpec((blk, 1), lambda i: (i, 0)),
        out_shape=jax.ShapeDtypeStruct((B, 1), jnp.float32),
    )(xg, w0a, w0b, b0r, W1, b1r, W2, b2r, wpa, wpb, bpr)
    return out.reshape(-1)


def kernel(user, item, embed_user_GMF, embed_item_GMF, embed_user_MLP,
           embed_item_MLP, W0, b0, W1, b1, W2, b2, Wp, bp):
    B = user.shape[0]
    u32 = user.astype(jnp.int32)
    i32 = item.astype(jnp.int32)
    eg, ig, em, im = _gather4(u32, i32, embed_user_GMF.T, embed_item_GMF.T,
                              embed_user_MLP.T, embed_item_MLP.T, B)
    xg = jnp.concatenate([eg, ig, em, im], axis=1)
    return _mlp_tc(xg, W0, b0, W1, b1, W2, b2, Wp, bp, B)


# speedup vs baseline: 1.3954x; 1.0016x over previous
import jax
import jax.numpy as jnp
from jax import lax
from jax.experimental import pallas as pl
from jax.experimental.pallas import tpu as pltpu

F = 32
CHUNK = 128
W = 128
NTILE = 7811
TAIL = 999872
TCUT = 999936


def _gather_body(u_sref, i_sref, t_ug, t_ig, t_um, t_im,
                 tl_ug, tl_ig, tl_um, tl_im,
                 o_ug, o_ig, o_um, o_im,
                 s_ug, s_ig, s_um, s_im, cm_u, cm_i,
                 sem_ug, sem_ig, sem_um, sem_im):
    ci = pl.program_id(0)
    srcs = (t_ug, t_ig, t_um, t_im)
    scrs = (s_ug, s_ig, s_um, s_im)
    sems = (sem_ug, sem_ig, sem_um, sem_im)

    def win(r):
        off = pl.multiple_of(jnp.minimum(r >> 7, NTILE) * 128, 128)
        return off, jnp.where(r >= TCUT, r - TAIL, r - off)

    for j in range(CHUNK):
        ru = u_sref[ci * CHUNK + j]
        ri = i_sref[ci * CHUNK + j]
        off_u, cu = win(ru)
        off_i, col_i = win(ri)
        for t in range(4):
            off = off_u if t % 2 == 0 else off_i
            pltpu.make_async_copy(
                srcs[t].at[:, pl.ds(off, W)],
                scrs[t].at[pl.ds(j * F, F), :],
                sems[t]).start()
        cm_u[pl.ds(j * F, F), :] = jnp.full((F, 1), cu, jnp.int32)
        cm_i[pl.ds(j * F, F), :] = jnp.full((F, 1), col_i, jnp.int32)

    for t in range(4):
        for j in range(CHUNK):
            pltpu.make_async_copy(
                srcs[t].at[:, pl.ds(0, W)],
                scrs[t].at[pl.ds(j * F, F), :],
                sems[t]).wait()

    for j in range(CHUNK):
        ru = u_sref[ci * CHUNK + j]
        ri = i_sref[ci * CHUNK + j]

        @pl.when(ru >= TCUT)
        def _():
            s_ug[pl.ds(j * F, F), :] = tl_ug[...]
            s_um[pl.ds(j * F, F), :] = tl_um[...]

        @pl.when(ri >= TCUT)
        def _():
            s_ig[pl.ds(j * F, F), :] = tl_ig[...]
            s_im[pl.ds(j * F, F), :] = tl_im[...]

    lane = lax.broadcasted_iota(jnp.int32, (CHUNK * F, W), 1)
    ones = jnp.ones((W, 1), jnp.float32)
    mask_u = lane == cm_u[...]
    mask_i = lane == cm_i[...]
    o_ug[...] = jnp.dot(jnp.where(mask_u, s_ug[...], 0.0), ones)
    o_ig[...] = jnp.dot(jnp.where(mask_i, s_ig[...], 0.0), ones)
    o_um[...] = jnp.dot(jnp.where(mask_u, s_um[...], 0.0), ones)
    o_im[...] = jnp.dot(jnp.where(mask_i, s_im[...], 0.0), ones)


def _gather4(user, item, t_ug, t_ig, t_um, t_im, B):
    n = B // CHUNK
    blk = CHUNK * F
    tbl_spec = pl.BlockSpec(memory_space=pl.ANY)
    tail_spec = pl.BlockSpec((F, W), lambda i, su, si: (0, 0))
    out_spec = pl.BlockSpec((blk, 1), lambda i, su, si: (i, 0))
    grid_spec = pltpu.PrefetchScalarGridSpec(
        num_scalar_prefetch=2,
        grid=(n,),
        in_specs=[tbl_spec] * 4 + [tail_spec] * 4,
        out_specs=[out_spec] * 4,
        scratch_shapes=(
            [pltpu.VMEM((blk, W), jnp.float32) for _ in range(4)]
            + [pltpu.VMEM((blk, 1), jnp.int32) for _ in range(2)]
            + [pltpu.SemaphoreType.DMA for _ in range(4)]
        ),
    )
    tails = [t[:, TAIL:] for t in (t_ug, t_ig, t_um, t_im)]
    outs = pl.pallas_call(
        _gather_body,
        grid_spec=grid_spec,
        out_shape=[jax.ShapeDtypeStruct((B * F, 1), jnp.float32)] * 4,
        compiler_params=pltpu.CompilerParams(
            dimension_semantics=("parallel",)),
    )(user, item, t_ug, t_ig, t_um, t_im, *tails)
    return [o.reshape(B, F) for o in outs]


def _mish(x):
    return x * jnp.tanh(jax.nn.softplus(x))


def _mlp_body(xr, w0a, w0b, b0r, w1, b1r, w2, b2r, wpa, wpb, bpr, out):
    x = xr[...]
    eu_m = x[:, 2 * F:3 * F]
    ei_m = x[:, 3 * F:4 * F]
    h = jnp.dot(eu_m, w0a[...]) + jnp.dot(ei_m, w0b[...]) + b0r[...]
    h = _mish(h)
    h = _mish(jnp.dot(h, w1[...]) + b1r[...])
    h = _mish(jnp.dot(h, w2[...]) + b2r[...])
    g = x[:, 0:F] * x[:, F:2 * F]
    p = (jnp.sum(g * wpa[...], axis=1, keepdims=True)
         + jnp.sum(h * wpb[...], axis=1, keepdims=True) + bpr[...])
    out[...] = _mish(p)


def _mlp_tc(xg, W0, b0, W1, b1, W2, b2, Wp, bp, B):
    blk = 2048
    grid = (B // blk,)
    w0a = W0[:F]
    w0b = W0[F:]
    wpa = Wp[:F].reshape(1, F)
    wpb = Wp[F:].reshape(1, F)
    b0r = b0.reshape(1, -1)
    b1r = b1.reshape(1, -1)
    b2r = b2.reshape(1, -1)
    bpr = bp.reshape(1, 1)

    def full_spec(a):
        return pl.BlockSpec(a.shape, lambda i: (0,) * a.ndim)

    out = pl.pallas_call(
        _mlp_body,
        grid=grid,
        in_specs=[
            pl.BlockSpec((blk, 4 * F), lambda i: (i, 0)),
            full_spec(w0a), full_spec(w0b), full_spec(b0r),
            full_spec(W1), full_spec(b1r),
            full_spec(W2), full_spec(b2r),
            full_spec(wpa), full_spec(wpb), full_spec(bpr),
        ],
        out_specs=pl.BlockSpec((blk, 1), lambda i: (i, 0)),
        out_shape=jax.ShapeDtypeStruct((B, 1), jnp.float32),
    )(xg, w0a, w0b, b0r, W1, b1r, W2, b2r, wpa, wpb, bpr)
    return out.reshape(-1)


def kernel(user, item, embed_user_GMF, embed_item_GMF, embed_user_MLP,
           embed_item_MLP, W0, b0, W1, b1, W2, b2, Wp, bp):
    B = user.shape[0]
    u32 = user.astype(jnp.int32)
    i32 = item.astype(jnp.int32)
    eg, ig, em, im = _gather4(u32, i32, embed_user_GMF.T, embed_item_GMF.T,
                              embed_user_MLP.T, embed_item_MLP.T, B)
    xg = jnp.concatenate([eg, ig, em, im], axis=1)
    return _mlp_tc(xg, W0, b0, W1, b1, W2, b2, Wp, bp, B)
